# baseline (speedup 1.0000x reference)
import functools

import jax
import jax.numpy as jnp
from jax import lax
from jax.experimental import pallas as pl
from jax.experimental.pallas import tpu as pltpu
from jax.experimental.pallas import tpu_sc as plsc

_DIM = 768
_E = 64
_K = 8
_N = 32768
_BLK = 256
_GRID = _N // _BLK

_NW = 32
_ENTRIES = _N * _K
_PER = _ENTRIES // _NW
_ROWS = _PER // 128


def _router_tc_body(x_ref, w_ref, tv_ref, te_ref, lp_ref, hist_ref,
                    offs_ref, rc_ref):
    i = pl.program_id(0)

    @pl.when(i == 0)
    def _():
        rc_ref[...] = jnp.zeros_like(rc_ref)

    x = x_ref[...]
    w = w_ref[...]
    scores = lax.dot_general(x, w, (((1,), (1,)), ((), ())),
                             preferred_element_type=jnp.float32)
    m = jnp.max(scores, axis=1, keepdims=True)
    ex = jnp.exp(scores - m)
    p = ex / jnp.sum(ex, axis=1, keepdims=True)

    iota_e = lax.broadcasted_iota(jnp.int32, (_BLK, _E), 1)
    vals, idxs, ohs = [], [], []
    s = p
    for _k in range(_K):
        mk = jnp.max(s, axis=1, keepdims=True)
        ik = jnp.min(jnp.where(s == mk, iota_e, _E), axis=1, keepdims=True)
        oh = iota_e == ik
        vals.append(mk)
        idxs.append(ik)
        ohs.append(oh.astype(jnp.float32))
        s = jnp.where(oh, -1.0, s)

    h = ohs[0]
    for _k in range(1, _K):
        h = h + ohs[_k]
    r_i = lax.broadcasted_iota(jnp.int32, (_BLK, _BLK), 0)
    c_i = lax.broadcasted_iota(jnp.int32, (_BLK, _BLK), 1)
    ltri = (c_i < r_i).astype(jnp.float32)
    pfx = lax.dot_general(ltri, h, (((1,), (0,)), ((), ())),
                          precision=lax.Precision.HIGHEST,
                          preferred_element_type=jnp.float32)
    q = pfx + rc_ref[...]

    lps = []
    for _k in range(_K):
        lps.append(jnp.sum(ohs[_k] * q, axis=1, keepdims=True))

    tv_ref[...] = jnp.concatenate(vals, axis=1)
    te_ref[...] = jnp.concatenate(idxs, axis=1).astype(jnp.int32)
    lp_ref[...] = jnp.concatenate(lps, axis=1).astype(jnp.int32)

    rc_new = rc_ref[...] + jnp.sum(h, axis=0, keepdims=True)
    rc_ref[...] = rc_new

    @pl.when(i == _GRID - 1)
    def _():
        hist_ref[...] = jnp.broadcast_to(rc_new, (8, _E))
        e_r = lax.broadcasted_iota(jnp.int32, (_E, _E), 0)
        e_c = lax.broadcasted_iota(jnp.int32, (_E, _E), 1)
        stri = (e_r < e_c).astype(jnp.float32)
        offs = lax.dot_general(rc_new, stri, (((1,), (0,)), ((), ())),
                               precision=lax.Precision.HIGHEST,
                               preferred_element_type=jnp.float32)
        offs_ref[...] = jnp.broadcast_to(offs, (8, _E)).astype(jnp.int32)


def _router_tc(x, w):
    return pl.pallas_call(
        _router_tc_body,
        grid=(_GRID,),
        in_specs=[
            pl.BlockSpec((_BLK, _DIM), lambda i: (i, 0)),
            pl.BlockSpec((_E, _DIM), lambda i: (0, 0)),
        ],
        out_specs=[
            pl.BlockSpec((_BLK, _K), lambda i: (i, 0)),
            pl.BlockSpec((_BLK, _K), lambda i: (i, 0)),
            pl.BlockSpec((_BLK, _K), lambda i: (i, 0)),
            pl.BlockSpec((8, _E), lambda i: (0, 0)),
            pl.BlockSpec((8, _E), lambda i: (0, 0)),
        ],
        out_shape=[
            jax.ShapeDtypeStruct((_N, _K), jnp.float32),
            jax.ShapeDtypeStruct((_N, _K), jnp.int32),
            jax.ShapeDtypeStruct((_N, _K), jnp.int32),
            jax.ShapeDtypeStruct((8, _E), jnp.float32),
            jax.ShapeDtypeStruct((8, _E), jnp.int32),
        ],
        scratch_shapes=[pltpu.VMEM((1, _E), jnp.float32)],
    )(x, w)


def _scatter_sc(vals2d, eids2d, lps2d, offs):
    mesh = plsc.VectorSubcoreMesh(core_axis_name="c", subcore_axis_name="s")

    @functools.partial(
        pl.kernel,
        mesh=mesh,
        compiler_params=pltpu.CompilerParams(needs_layout_passes=False),
        out_type=[
            jax.ShapeDtypeStruct((_ENTRIES,), jnp.float32),
            jax.ShapeDtypeStruct((_ENTRIES,), jnp.int32),
        ],
        scratch_types=[
            pltpu.VMEM((_PER,), jnp.int32),
            pltpu.VMEM((_PER,), jnp.int32),
            pltpu.VMEM((_PER,), jnp.float32),
            pltpu.VMEM((_PER,), jnp.int32),
            pltpu.VMEM((_PER,), jnp.int32),
            pltpu.VMEM((_E,), jnp.int32),
            pltpu.SemaphoreType.DMA,
        ],
    )
    def k(vals_hbm, eids_hbm, lps_hbm, offs_hbm, out_s_hbm, out_t_hbm,
          e_v, lp_v, val_v, pos_v, tok_v, off_v, sem):
        wid = lax.axis_index("s") * 2 + lax.axis_index("c")
        gbase = wid * _PER
        pltpu.sync_copy(offs_hbm, off_v)
        pltpu.sync_copy(eids_hbm.at[pl.ds(gbase, _PER)], e_v)
        pltpu.sync_copy(lps_hbm.at[pl.ds(gbase, _PER)], lp_v)
        pltpu.sync_copy(vals_hbm.at[pl.ds(gbase, _PER)], val_v)

        lane = lax.broadcasted_iota(jnp.int32, (16,), 0)

        def chunk_body(r, carry):
            base = r * 128
            for c in range(8):
                sl = pl.ds(base + c * 16, 16)
                e16 = e_v[sl]
                lp16 = lp_v[sl]
                off16 = plsc.load_gather(off_v, [e16])
                pos_v[sl] = off16 + lp16
                ent = gbase + base + c * 16 + lane
                tok_v[sl] = lax.shift_right_logical(ent, 3)
            return carry

        lax.fori_loop(0, _ROWS, chunk_body, 0)

        cp1 = pltpu.make_async_copy(val_v, out_s_hbm.at[pos_v], sem)
        cp2 = pltpu.make_async_copy(tok_v, out_t_hbm.at[pos_v], sem)
        cp1.start()
        cp2.start()
        cp1.wait()
        cp2.wait()

    return k(vals2d, eids2d, lps2d, offs)


def kernel(x, W):
    tv, te, lp, hist8, offs8 = _router_tc(x, W)
    out_s, out_t = _scatter_sc(tv.reshape(_ENTRIES), te.reshape(_ENTRIES),
                               lp.reshape(_ENTRIES), offs8[0])
    return out_s, out_t, hist8[0]


# speedup vs baseline: 1.9919x; 1.9919x over previous
import functools

import jax
import jax.numpy as jnp
from jax import lax
from jax.experimental import pallas as pl
from jax.experimental.pallas import tpu as pltpu
from jax.experimental.pallas import tpu_sc as plsc

_DIM = 768
_E = 64
_K = 8
_N = 32768
_BLK = 256
_GRID = _N // _BLK

_NW = 32
_ENTRIES = _N * _K
_PER = _ENTRIES // _NW
_ROWS = _PER // 128


def _router_tc_body(x_ref, w_ref, tv_ref, te_ref, lp_ref, hist_ref,
                    offs_ref, rc_ref):
    i = pl.program_id(0)

    @pl.when(i == 0)
    def _():
        rc_ref[...] = jnp.zeros_like(rc_ref)

    x = x_ref[...]
    w = w_ref[...]
    scores = lax.dot_general(x, w, (((1,), (1,)), ((), ())),
                             preferred_element_type=jnp.float32)
    m = jnp.max(scores, axis=1, keepdims=True)
    ex = jnp.exp(scores - m)
    p = ex / jnp.sum(ex, axis=1, keepdims=True)

    iota_e = lax.broadcasted_iota(jnp.int32, (_BLK, _E), 1)
    vals, idxs, ohs = [], [], []
    s = p
    for _k in range(_K):
        mk = jnp.max(s, axis=1, keepdims=True)
        ik = jnp.min(jnp.where(s == mk, iota_e, _E), axis=1, keepdims=True)
        oh = iota_e == ik
        vals.append(mk)
        idxs.append(ik)
        ohs.append(oh.astype(jnp.float32))
        s = jnp.where(oh, -1.0, s)

    h = ohs[0]
    for _k in range(1, _K):
        h = h + ohs[_k]
    r_i = lax.broadcasted_iota(jnp.int32, (_BLK, _BLK), 0)
    c_i = lax.broadcasted_iota(jnp.int32, (_BLK, _BLK), 1)
    ltri = (c_i < r_i).astype(jnp.float32)
    pfx = lax.dot_general(ltri, h, (((1,), (0,)), ((), ())),
                          precision=lax.Precision.HIGHEST,
                          preferred_element_type=jnp.float32)
    q = pfx + rc_ref[...]

    lps = []
    for _k in range(_K):
        lps.append(jnp.sum(ohs[_k] * q, axis=1, keepdims=True))

    tv_ref[...] = jnp.concatenate(vals, axis=1)
    te_ref[...] = jnp.concatenate(idxs, axis=1).astype(jnp.int32)
    lp_ref[...] = jnp.concatenate(lps, axis=1).astype(jnp.int32)

    rc_new = rc_ref[...] + jnp.sum(h, axis=0, keepdims=True)
    rc_ref[...] = rc_new

    @pl.when(i == _GRID - 1)
    def _():
        hist_ref[...] = jnp.broadcast_to(rc_new, (8, _E))
        e_r = lax.broadcasted_iota(jnp.int32, (_E, _E), 0)
        e_c = lax.broadcasted_iota(jnp.int32, (_E, _E), 1)
        stri = (e_r < e_c).astype(jnp.float32)
        offs = lax.dot_general(rc_new, stri, (((1,), (0,)), ((), ())),
                               precision=lax.Precision.HIGHEST,
                               preferred_element_type=jnp.float32)
        offs_ref[...] = jnp.broadcast_to(offs, (8, _E)).astype(jnp.int32)


def _router_tc(x, w):
    return pl.pallas_call(
        _router_tc_body,
        grid=(_GRID,),
        in_specs=[
            pl.BlockSpec((_BLK, _DIM), lambda i: (i, 0)),
            pl.BlockSpec((_E, _DIM), lambda i: (0, 0)),
        ],
        out_specs=[
            pl.BlockSpec((_BLK, _K), lambda i: (i, 0)),
            pl.BlockSpec((_BLK, _K), lambda i: (i, 0)),
            pl.BlockSpec((_BLK, _K), lambda i: (i, 0)),
            pl.BlockSpec((8, _E), lambda i: (0, 0)),
            pl.BlockSpec((8, _E), lambda i: (0, 0)),
        ],
        out_shape=[
            jax.ShapeDtypeStruct((_N, _K), jnp.float32),
            jax.ShapeDtypeStruct((_N, _K), jnp.int32),
            jax.ShapeDtypeStruct((_N, _K), jnp.int32),
            jax.ShapeDtypeStruct((8, _E), jnp.float32),
            jax.ShapeDtypeStruct((8, _E), jnp.int32),
        ],
        scratch_shapes=[pltpu.VMEM((1, _E), jnp.float32)],
    )(x, w)


_HALF = _ENTRIES // 2
_TPER = _ENTRIES // 16
_OUTW = _HALF // 16


def _scatter_sc(vals, eids, lps, offs):
    mesh = plsc.VectorSubcoreMesh(core_axis_name="c", subcore_axis_name="s")

    @functools.partial(
        pl.kernel,
        mesh=mesh,
        compiler_params=pltpu.CompilerParams(needs_layout_passes=False),
        out_type=[
            jax.ShapeDtypeStruct((_ENTRIES,), jnp.float32),
            jax.ShapeDtypeStruct((_ENTRIES,), jnp.int32),
        ],
        scratch_types=[
            pltpu.VMEM((_TPER,), jnp.int32),
            pltpu.VMEM((_TPER,), jnp.int32),
            pltpu.VMEM((_TPER,), jnp.float32),
            pltpu.VMEM((_TPER,), jnp.int32),
            pltpu.VMEM((_TPER,), jnp.int32),
            pltpu.VMEM((_E,), jnp.int32),
            pltpu.VMEM_SHARED((_HALF + 16,), jnp.float32),
            pltpu.VMEM_SHARED((_HALF + 16,), jnp.int32),
            pltpu.SemaphoreType.DMA,
        ],
    )
    def k(vals_hbm, eids_hbm, lps_hbm, offs_hbm, out_s_hbm, out_t_hbm,
          e_v, lp_v, val_v, pos_v, tok_v, off_v, s_sp, t_sp, sem):
        cid = lax.axis_index("c")
        sid = lax.axis_index("s")
        base = sid * _TPER
        half0 = cid * _HALF
        pltpu.sync_copy(offs_hbm, off_v)
        pltpu.sync_copy(eids_hbm.at[pl.ds(base, _TPER)], e_v)
        pltpu.sync_copy(lps_hbm.at[pl.ds(base, _TPER)], lp_v)
        pltpu.sync_copy(vals_hbm.at[pl.ds(base, _TPER)], val_v)

        lane = lax.broadcasted_iota(jnp.int32, (16,), 0)
        trash = _HALF + lane

        def chunk_body(r, carry):
            for c in range(8):
                sl = pl.ds(r * 128 + c * 16, 16)
                e16 = e_v[sl]
                lp16 = lp_v[sl]
                p = plsc.load_gather(off_v, [e16]) + lp16 - half0
                inside = (p >= 0) & (p < _HALF)
                pos_v[sl] = jnp.where(inside, p, trash)
                ent = base + r * 128 + c * 16 + lane
                tok_v[sl] = lax.shift_right_logical(ent, 3)
            return carry

        lax.fori_loop(0, _TPER // 128, chunk_body, 0)

        cp1 = pltpu.make_async_copy(val_v, s_sp.at[pos_v], sem)
        cp2 = pltpu.make_async_copy(tok_v, t_sp.at[pos_v], sem)
        cp1.start()
        cp2.start()
        cp1.wait()
        cp2.wait()
        plsc.subcore_barrier()

        osl = pl.ds(sid * _OUTW, _OUTW)
        gsl = pl.ds(half0 + sid * _OUTW, _OUTW)
        pltpu.sync_copy(s_sp.at[osl], out_s_hbm.at[gsl])
        pltpu.sync_copy(t_sp.at[osl], out_t_hbm.at[gsl])

    return k(vals, eids, lps, offs)


def kernel(x, W):
    tv, te, lp, hist8, offs8 = _router_tc(x, W)
    out_s, out_t = _scatter_sc(tv.reshape(_ENTRIES), te.reshape(_ENTRIES),
                               lp.reshape(_ENTRIES), offs8[0])
    return out_s, out_t, hist8[0]


# speedup vs baseline: 4.4959x; 2.2571x over previous
import functools

import jax
import jax.numpy as jnp
from jax import lax
from jax.experimental import pallas as pl
from jax.experimental.pallas import tpu as pltpu
from jax.experimental.pallas import tpu_sc as plsc

_DIM = 768
_E = 64
_K = 8
_N = 32768
_BLK = 256
_GRID = _N // _BLK

_NW = 32
_ENTRIES = _N * _K
_PER = _ENTRIES // _NW
_ROWS = _PER // 128


def _router_tc_body(x_ref, w_ref, tv_ref, te_ref, lp_ref, hist_ref,
                    offs_ref, rc_ref, u_ref):
    i = pl.program_id(0)

    @pl.when(i == 0)
    def _():
        rc_ref[...] = jnp.zeros_like(rc_ref)
        r_i = lax.broadcasted_iota(jnp.int32, (_BLK, _BLK), 0)
        c_i = lax.broadcasted_iota(jnp.int32, (_BLK, _BLK), 1)
        u_ref[...] = (r_i < c_i).astype(jnp.float32)

    scores = lax.dot_general(w_ref[...], x_ref[...], (((1,), (1,)), ((), ())),
                             preferred_element_type=jnp.float32)
    m = jnp.max(scores, axis=0, keepdims=True)
    ex = jnp.exp(scores - m)
    p = ex / jnp.sum(ex, axis=0, keepdims=True)

    iota_e = lax.broadcasted_iota(jnp.int32, (_E, _BLK), 0)
    vals, idxs, ohs = [], [], []
    s = p
    for _k in range(_K):
        mk = jnp.max(s, axis=0, keepdims=True)
        ik = jnp.min(jnp.where(s == mk, iota_e, _E), axis=0, keepdims=True)
        oh = iota_e == ik
        vals.append(mk)
        idxs.append(ik)
        ohs.append(oh)
        s = jnp.where(oh, -1.0, s)

    h = (s == -1.0).astype(jnp.float32)
    pfx = lax.dot_general(h, u_ref[...], (((1,), (0,)), ((), ())),
                          precision=lax.Precision.HIGHEST,
                          preferred_element_type=jnp.float32)
    q = pfx + rc_ref[...]

    lps = []
    for _k in range(_K):
        lps.append(jnp.sum(jnp.where(ohs[_k], q, 0.0), axis=0, keepdims=True))

    tv_ref[...] = jnp.concatenate(vals, axis=0)
    te_ref[...] = jnp.concatenate(idxs, axis=0)
    lp_ref[...] = jnp.concatenate(lps, axis=0).astype(jnp.int32)

    rc_new = rc_ref[...] + jnp.sum(h, axis=1, keepdims=True)
    rc_ref[...] = rc_new

    @pl.when(i == _GRID - 1)
    def _():
        hist_ref[...] = jnp.broadcast_to(rc_new, (_E, 128))
        e_r = lax.broadcasted_iota(jnp.int32, (_E, _E), 0)
        e_c = lax.broadcasted_iota(jnp.int32, (_E, _E), 1)
        stri = (e_c < e_r).astype(jnp.float32)
        offs = lax.dot_general(stri, rc_new, (((1,), (0,)), ((), ())),
                               precision=lax.Precision.HIGHEST,
                               preferred_element_type=jnp.float32)
        offs_ref[...] = jnp.broadcast_to(offs, (_E, 128)).astype(jnp.int32)


def _router_tc(x, w):
    return pl.pallas_call(
        _router_tc_body,
        grid=(_GRID,),
        in_specs=[
            pl.BlockSpec((_BLK, _DIM), lambda i: (i, 0)),
            pl.BlockSpec((_E, _DIM), lambda i: (0, 0)),
        ],
        out_specs=[
            pl.BlockSpec((_K, _BLK), lambda i: (0, i)),
            pl.BlockSpec((_K, _BLK), lambda i: (0, i)),
            pl.BlockSpec((_K, _BLK), lambda i: (0, i)),
            pl.BlockSpec((_E, 128), lambda i: (0, 0)),
            pl.BlockSpec((_E, 128), lambda i: (0, 0)),
        ],
        out_shape=[
            jax.ShapeDtypeStruct((_K, _N), jnp.float32),
            jax.ShapeDtypeStruct((_K, _N), jnp.int32),
            jax.ShapeDtypeStruct((_K, _N), jnp.int32),
            jax.ShapeDtypeStruct((_E, 128), jnp.float32),
            jax.ShapeDtypeStruct((_E, 128), jnp.int32),
        ],
        scratch_shapes=[pltpu.VMEM((_E, 1), jnp.float32),
                        pltpu.VMEM((_BLK, _BLK), jnp.float32)],
    )(x, w)


_HALF = _ENTRIES // 2
_TPER = _ENTRIES // 16
_OUTW = _HALF // 16


def _scatter_sc(vals, eids, lps, offs):
    mesh = plsc.VectorSubcoreMesh(core_axis_name="c", subcore_axis_name="s")

    @functools.partial(
        pl.kernel,
        mesh=mesh,
        compiler_params=pltpu.CompilerParams(needs_layout_passes=False),
        out_type=[
            jax.ShapeDtypeStruct((_ENTRIES,), jnp.float32),
            jax.ShapeDtypeStruct((_ENTRIES,), jnp.int32),
        ],
        scratch_types=[
            pltpu.VMEM((_TPER,), jnp.int32),
            pltpu.VMEM((_TPER,), jnp.int32),
            pltpu.VMEM((_TPER,), jnp.float32),
            pltpu.VMEM((_TPER,), jnp.int32),
            pltpu.VMEM((_TPER,), jnp.int32),
            pltpu.VMEM((_E,), jnp.int32),
            pltpu.VMEM_SHARED((_HALF + 16,), jnp.float32),
            pltpu.VMEM_SHARED((_HALF + 16,), jnp.int32),
            pltpu.SemaphoreType.DMA,
        ],
    )
    def k(vals_hbm, eids_hbm, lps_hbm, offs_hbm, out_s_hbm, out_t_hbm,
          e_v, lp_v, val_v, pos_v, tok_v, off_v, s_sp, t_sp, sem):
        cid = lax.axis_index("c")
        sid = lax.axis_index("s")
        base = sid * _TPER
        half0 = cid * _HALF
        pltpu.sync_copy(offs_hbm, off_v)
        pltpu.sync_copy(eids_hbm.at[pl.ds(base, _TPER)], e_v)
        pltpu.sync_copy(lps_hbm.at[pl.ds(base, _TPER)], lp_v)
        pltpu.sync_copy(vals_hbm.at[pl.ds(base, _TPER)], val_v)

        lane = lax.broadcasted_iota(jnp.int32, (16,), 0)
        trash = _HALF + lane

        def chunk_body(r, carry):
            for c in range(8):
                sl = pl.ds(r * 128 + c * 16, 16)
                e16 = e_v[sl]
                lp16 = lp_v[sl]
                p = plsc.load_gather(off_v, [e16]) + lp16 - half0
                inside = (p >= 0) & (p < _HALF)
                pos_v[sl] = jnp.where(inside, p, trash)
                ent = base + r * 128 + c * 16 + lane
                tok_v[sl] = jnp.bitwise_and(ent, _N - 1)
            return carry

        lax.fori_loop(0, _TPER // 128, chunk_body, 0)

        cp1 = pltpu.make_async_copy(val_v, s_sp.at[pos_v], sem)
        cp2 = pltpu.make_async_copy(tok_v, t_sp.at[pos_v], sem)
        cp1.start()
        cp2.start()
        cp1.wait()
        cp2.wait()
        plsc.subcore_barrier()

        osl = pl.ds(sid * _OUTW, _OUTW)
        gsl = pl.ds(half0 + sid * _OUTW, _OUTW)
        pltpu.sync_copy(s_sp.at[osl], out_s_hbm.at[gsl])
        pltpu.sync_copy(t_sp.at[osl], out_t_hbm.at[gsl])

    return k(vals, eids, lps, offs)


def kernel(x, W):
    tv, te, lp, hist2d, offs2d = _router_tc(x, W)
    out_s, out_t = _scatter_sc(tv.reshape(_ENTRIES), te.reshape(_ENTRIES),
                               lp.reshape(_ENTRIES), offs2d[:, 0])
    return out_s, out_t, hist2d[:, 0]


# speedup vs baseline: 5.2980x; 1.1784x over previous
import functools

import jax
import jax.numpy as jnp
from jax import lax
from jax.experimental import pallas as pl
from jax.experimental.pallas import tpu as pltpu
from jax.experimental.pallas import tpu_sc as plsc

_DIM = 768
_E = 64
_K = 8
_N = 32768
_BLK = 512
_GRID = _N // _BLK

_NW = 32
_ENTRIES = _N * _K
_PER = _ENTRIES // _NW
_ROWS = _PER // 128


def _router_tc_body(x_ref, w_ref, tv_ref, te_ref, lp_ref, hist_ref,
                    offs_ref, rc_ref, u_ref):
    i = pl.program_id(0)

    @pl.when(i == 0)
    def _():
        rc_ref[...] = jnp.zeros_like(rc_ref)
        r_i = lax.broadcasted_iota(jnp.int32, (_BLK, _BLK), 0)
        c_i = lax.broadcasted_iota(jnp.int32, (_BLK, _BLK), 1)
        u_ref[...] = (r_i < c_i).astype(jnp.float32)

    scores = lax.dot_general(w_ref[...], x_ref[...], (((1,), (1,)), ((), ())),
                             preferred_element_type=jnp.float32)
    m = jnp.max(scores, axis=0, keepdims=True)
    ex = jnp.exp(scores - m)
    p = ex / jnp.sum(ex, axis=0, keepdims=True)

    iota_e = lax.broadcasted_iota(jnp.int32, (_E, _BLK), 0)
    vals, idxs, ohs = [], [], []
    s = p
    for _k in range(_K):
        mk = jnp.max(s, axis=0, keepdims=True)
        ik = jnp.min(jnp.where(s == mk, iota_e, _E), axis=0, keepdims=True)
        oh = iota_e == ik
        vals.append(mk)
        idxs.append(ik)
        ohs.append(oh)
        s = jnp.where(oh, -1.0, s)

    h = (s == -1.0).astype(jnp.float32)
    pfx = lax.dot_general(h, u_ref[...], (((1,), (0,)), ((), ())),
                          precision=lax.Precision.HIGHEST,
                          preferred_element_type=jnp.float32)
    q = pfx + rc_ref[...]

    lps = []
    for _k in range(_K):
        lps.append(jnp.sum(jnp.where(ohs[_k], q, 0.0), axis=0, keepdims=True))

    tv_ref[...] = jnp.concatenate(vals, axis=0)
    te_ref[...] = jnp.concatenate(idxs, axis=0)
    lp_ref[...] = jnp.concatenate(lps, axis=0).astype(jnp.int32)

    rc_new = rc_ref[...] + jnp.sum(h, axis=1, keepdims=True)
    rc_ref[...] = rc_new

    @pl.when(i == _GRID - 1)
    def _():
        hist_ref[...] = jnp.broadcast_to(rc_new, (_E, 128))
        e_r = lax.broadcasted_iota(jnp.int32, (_E, _E), 0)
        e_c = lax.broadcasted_iota(jnp.int32, (_E, _E), 1)
        stri = (e_c < e_r).astype(jnp.float32)
        offs = lax.dot_general(stri, rc_new, (((1,), (0,)), ((), ())),
                               precision=lax.Precision.HIGHEST,
                               preferred_element_type=jnp.float32)
        offs_ref[...] = jnp.broadcast_to(offs, (_E, 128)).astype(jnp.int32)


def _router_tc(x, w):
    return pl.pallas_call(
        _router_tc_body,
        grid=(_GRID,),
        in_specs=[
            pl.BlockSpec((_BLK, _DIM), lambda i: (i, 0)),
            pl.BlockSpec((_E, _DIM), lambda i: (0, 0)),
        ],
        out_specs=[
            pl.BlockSpec((_K, _BLK), lambda i: (0, i)),
            pl.BlockSpec((_K, _BLK), lambda i: (0, i)),
            pl.BlockSpec((_K, _BLK), lambda i: (0, i)),
            pl.BlockSpec((_E, 128), lambda i: (0, 0)),
            pl.BlockSpec((_E, 128), lambda i: (0, 0)),
        ],
        out_shape=[
            jax.ShapeDtypeStruct((_K, _N), jnp.float32),
            jax.ShapeDtypeStruct((_K, _N), jnp.int32),
            jax.ShapeDtypeStruct((_K, _N), jnp.int32),
            jax.ShapeDtypeStruct((_E, 128), jnp.float32),
            jax.ShapeDtypeStruct((_E, 128), jnp.int32),
        ],
        scratch_shapes=[pltpu.VMEM((_E, 1), jnp.float32),
                        pltpu.VMEM((_BLK, _BLK), jnp.float32)],
    )(x, w)


_HALF = _ENTRIES // 2
_TPER = _ENTRIES // 16
_OUTW = _HALF // 16


def _scatter_sc(vals, eids, lps, offs):
    mesh = plsc.VectorSubcoreMesh(core_axis_name="c", subcore_axis_name="s")

    @functools.partial(
        pl.kernel,
        mesh=mesh,
        compiler_params=pltpu.CompilerParams(needs_layout_passes=False),
        out_type=[
            jax.ShapeDtypeStruct((_ENTRIES,), jnp.float32),
            jax.ShapeDtypeStruct((_ENTRIES,), jnp.int32),
        ],
        scratch_types=[
            pltpu.VMEM((_TPER,), jnp.int32),
            pltpu.VMEM((_TPER,), jnp.int32),
            pltpu.VMEM((_TPER,), jnp.float32),
            pltpu.VMEM((_TPER,), jnp.int32),
            pltpu.VMEM((_TPER,), jnp.int32),
            pltpu.VMEM((_E,), jnp.int32),
            pltpu.VMEM_SHARED((_HALF + 16,), jnp.float32),
            pltpu.VMEM_SHARED((_HALF + 16,), jnp.int32),
            pltpu.SemaphoreType.DMA,
        ],
    )
    def k(vals_hbm, eids_hbm, lps_hbm, offs_hbm, out_s_hbm, out_t_hbm,
          e_v, lp_v, val_v, pos_v, tok_v, off_v, s_sp, t_sp, sem):
        cid = lax.axis_index("c")
        sid = lax.axis_index("s")
        base = sid * _TPER
        half0 = cid * _HALF
        pltpu.sync_copy(offs_hbm, off_v)
        pltpu.sync_copy(eids_hbm.at[pl.ds(base, _TPER)], e_v)
        pltpu.sync_copy(lps_hbm.at[pl.ds(base, _TPER)], lp_v)
        pltpu.sync_copy(vals_hbm.at[pl.ds(base, _TPER)], val_v)

        lane = lax.broadcasted_iota(jnp.int32, (16,), 0)
        trash = _HALF + lane

        def chunk_body(r, carry):
            for c in range(8):
                sl = pl.ds(r * 128 + c * 16, 16)
                e16 = e_v[sl]
                lp16 = lp_v[sl]
                p = plsc.load_gather(off_v, [e16]) + lp16 - half0
                inside = (p >= 0) & (p < _HALF)
                pos_v[sl] = jnp.where(inside, p, trash)
                ent = base + r * 128 + c * 16 + lane
                tok_v[sl] = jnp.bitwise_and(ent, _N - 1)
            return carry

        lax.fori_loop(0, _TPER // 128, chunk_body, 0)

        cp1 = pltpu.make_async_copy(val_v, s_sp.at[pos_v], sem)
        cp2 = pltpu.make_async_copy(tok_v, t_sp.at[pos_v], sem)
        cp1.start()
        cp2.start()
        cp1.wait()
        cp2.wait()
        plsc.subcore_barrier()

        osl = pl.ds(sid * _OUTW, _OUTW)
        gsl = pl.ds(half0 + sid * _OUTW, _OUTW)
        pltpu.sync_copy(s_sp.at[osl], out_s_hbm.at[gsl])
        pltpu.sync_copy(t_sp.at[osl], out_t_hbm.at[gsl])

    return k(vals, eids, lps, offs)


def kernel(x, W):
    tv, te, lp, hist2d, offs2d = _router_tc(x, W)
    out_s, out_t = _scatter_sc(tv.reshape(_ENTRIES), te.reshape(_ENTRIES),
                               lp.reshape(_ENTRIES), offs2d[:, 0])
    return out_s, out_t, hist2d[:, 0]
